# baseline (speedup 1.0000x reference)
import functools

import jax
import jax.numpy as jnp
from jax import lax
from jax.experimental import pallas as pl
from jax.experimental.pallas import tpu as pltpu
from jax.experimental.pallas import tpu_sc as plsc

N = 10000
NP = 10240
E = 320000
F = 128
CH = 128
NCH = 2560

f32 = jnp.float32
i32 = jnp.int32

_MESH = plsc.VectorSubcoreMesh(core_axis_name="c", subcore_axis_name="s")


def _deg_body(ei_hbm, dego_hbm, degi_hbm, slab, ones_v, zbuf, acc, sem):
    c = lax.axis_index("c")
    s = lax.axis_index("s")

    for u in range(8):
        ones_v[pl.ds(u * 16, 16)] = jnp.full((16,), 1.0, f32)
    for u in range(40):
        zbuf[pl.ds(u * 16, 16)] = jnp.zeros((16,), f32)

    pltpu.sync_copy(ei_hbm.at[c, pl.ds(s * 160, 160)], slab)
    pltpu.sync_copy(zbuf, acc.at[pl.ds(s * 640, 640)])

    plsc.subcore_barrier()

    def group(g, carry):
        for u in range(8):
            pltpu.async_copy(ones_v, acc.at[slab.at[g * 8 + u]], sem,
                             add=True)

        @pl.when(g > 0)
        def _():
            for _u in range(8):
                pltpu.make_async_copy(ones_v, acc.at[slab.at[0]],
                                      sem).wait()
        return carry

    lax.fori_loop(0, 20, group, 0)
    for _u in range(8):
        pltpu.make_async_copy(ones_v, acc.at[slab.at[0]], sem).wait()

    plsc.subcore_barrier()

    @pl.when(s < 15)
    def _():
        pltpu.sync_copy(acc.at[pl.ds(s * 640, 640)], zbuf)

        @pl.when(c == 0)
        def _():
            pltpu.sync_copy(zbuf, dego_hbm.at[pl.ds(s * 640, 640)])

        @pl.when(c == 1)
        def _():
            pltpu.sync_copy(zbuf, degi_hbm.at[pl.ds(s * 640, 640)])

    @pl.when(s == 15)
    def _():
        pltpu.sync_copy(acc.at[pl.ds(9600, 400)], zbuf.at[pl.ds(0, 400)])

        @pl.when(c == 0)
        def _():
            pltpu.sync_copy(zbuf.at[pl.ds(0, 400)],
                            dego_hbm.at[pl.ds(9600, 400)])

        @pl.when(c == 1)
        def _():
            pltpu.sync_copy(zbuf.at[pl.ds(0, 400)],
                            degi_hbm.at[pl.ds(9600, 400)])


_deg_call = functools.partial(
    pl.kernel,
    out_type=(jax.ShapeDtypeStruct((N,), f32),
              jax.ShapeDtypeStruct((N,), f32)),
    mesh=_MESH,
    scratch_types=[
        pltpu.VMEM((160, CH), i32),
        pltpu.VMEM((CH,), f32),
        pltpu.VMEM((640,), f32),
        pltpu.VMEM_SHARED((NP,), f32),
        pltpu.SemaphoreType.DMA,
    ],
)(_deg_body)


def _mp_body(hh_hbm, eq_hbm, out_hbm,
             sidx, didx, rows0, rows1, rows2, rows3,
             acc, g0, g1, g2, g3, s0, s1, s2, s3, lsem):
    c = lax.axis_index("c")
    s = lax.axis_index("s")
    q0 = c * 2048 + s * 128

    rows = (rows0, rows1, rows2, rows3)
    gsem = (g0, g1, g2, g3)
    ssem = (s0, s1, s2, s3)

    def zrow(i, carry):
        for u in range(8):
            rows0[i, pl.ds(u * 16, 16)] = jnp.zeros((16,), f32)
        return carry

    lax.fori_loop(0, 80, zrow, 0)
    for k in range(8):
        pltpu.async_copy(rows0, acc.at[pl.ds(s * 640 + k * 80, 80), :],
                         gsem[k % 4])
    for k in range(8):
        pltpu.make_async_copy(rows0, acc.at[pl.ds(0, 80), :],
                              gsem[k % 4]).wait()

    pltpu.sync_copy(eq_hbm.at[0, pl.ds(q0, 16)], sidx.at[0])
    pltpu.sync_copy(eq_hbm.at[1, pl.ds(q0, 16)], didx.at[0])
    plsc.subcore_barrier()

    def gather(slot, pb, r):
        pltpu.async_copy(hh_hbm.at[sidx.at[pb, r, pl.ds(0, 40)]],
                         rows[slot].at[pl.ds(0, 40), :], gsem[slot])
        pltpu.async_copy(hh_hbm.at[sidx.at[pb, r, pl.ds(40, 40)]],
                         rows[slot].at[pl.ds(40, 40), :], gsem[slot])

    def scatter_chunk(u2, tm2):
        pb2 = (tm2 // 16) % 2
        r2 = tm2 % 16
        pltpu.make_async_copy(hh_hbm.at[sidx.at[0, 0]], rows[u2],
                              gsem[u2]).wait()
        pltpu.async_copy(rows[u2], acc.at[didx.at[pb2, r2]],
                         ssem[u2], add=True)

    def quad(p):
        def body(qq, carry):
            for u in range(4):
                q = qq * 4 + u
                t = p * 16 + q

                @pl.when(t >= 4)
                def _():
                    pltpu.make_async_copy(rows[u], acc.at[didx.at[0, 0]],
                                          ssem[u]).wait()

                gather(u, p % 2, q)

                @pl.when(t >= 2)
                def _():
                    scatter_chunk((u + 2) % 4, t - 2)
            return carry
        return body

    for p in range(8):
        if p > 0:
            pltpu.make_async_copy(eq_hbm.at[0, pl.ds(q0, 16)],
                                  sidx.at[p % 2], lsem).wait()
            pltpu.make_async_copy(eq_hbm.at[1, pl.ds(q0, 16)],
                                  didx.at[p % 2], lsem).wait()
        lax.fori_loop(0, 2, quad(p), 0)
        if p < 7:
            nb = (p + 1) % 2
            pltpu.async_copy(eq_hbm.at[0, pl.ds(q0 + (p + 1) * 16, 16)],
                             sidx.at[nb], lsem)
            pltpu.async_copy(eq_hbm.at[1, pl.ds(q0 + (p + 1) * 16, 16)],
                             didx.at[nb], lsem)
        lax.fori_loop(2, 4, quad(p), 0)

    for t in (128, 129):
        scatter_chunk((t - 2) % 4, t - 2)
    for u in range(4):
        pltpu.make_async_copy(rows[u], acc.at[didx.at[0, 0]],
                              ssem[u]).wait()

    plsc.subcore_barrier()
    for k in range(8):
        buf = rows[k % 4]
        if k >= 4:
            pltpu.make_async_copy(
                buf, out_hbm.at[c, pl.ds(0, 80), :], gsem[k % 4]).wait()
        pltpu.sync_copy(acc.at[pl.ds(s * 640 + k * 80, 80), :], buf)
        pltpu.async_copy(buf, out_hbm.at[c, pl.ds(s * 640 + k * 80, 80), :],
                         gsem[k % 4])
    for k in range(4):
        pltpu.make_async_copy(rows[k], out_hbm.at[c, pl.ds(0, 80), :],
                              gsem[k]).wait()


_mp_call = functools.partial(
    pl.kernel,
    out_type=jax.ShapeDtypeStruct((2, NP, F), f32),
    mesh=_MESH,
    scratch_types=[
        pltpu.VMEM((2, 16, 80), i32),
        pltpu.VMEM((2, 16, 80), i32),
        pltpu.VMEM((80, F), f32),
        pltpu.VMEM((80, F), f32),
        pltpu.VMEM((80, F), f32),
        pltpu.VMEM((80, F), f32),
        pltpu.VMEM_SHARED((NP, F), f32),
        pltpu.SemaphoreType.DMA,
        pltpu.SemaphoreType.DMA,
        pltpu.SemaphoreType.DMA,
        pltpu.SemaphoreType.DMA,
        pltpu.SemaphoreType.DMA,
        pltpu.SemaphoreType.DMA,
        pltpu.SemaphoreType.DMA,
        pltpu.SemaphoreType.DMA,
        pltpu.SemaphoreType.DMA,
    ],
)(_mp_body)


_R = 2048


def _pre_body(h_ref, degt_ref, out_ref):
    ns = lax.rsqrt(jnp.clip(degt_ref[:, 0:1], 1.0, None))
    out_ref[...] = h_ref[...] * ns


def _tc_pre(h, degt):
    return pl.pallas_call(
        _pre_body,
        grid=(NP // _R,),
        in_specs=[
            pl.BlockSpec((_R, F), lambda i: (i, 0)),
            pl.BlockSpec((_R, 2), lambda i: (i, 0)),
        ],
        out_specs=pl.BlockSpec((_R, F), lambda i: (i, 0)),
        out_shape=jax.ShapeDtypeStruct((NP, F), f32),
    )(h, degt)


def _layer_body(apply_ns, agg_ref, degt_ref, w_ref, b_ref, out_ref):
    a = agg_ref[0] + agg_ref[1]
    d = degt_ref[...]
    nd = lax.rsqrt(jnp.clip(d[:, 1:2], 1.0, None))
    x = jnp.dot(a * nd, w_ref[...], preferred_element_type=f32)
    x = jnp.maximum(x + b_ref[...], 0.0)
    if apply_ns:
        x = x * lax.rsqrt(jnp.clip(d[:, 0:1], 1.0, None))
    out_ref[...] = x


def _tc_layer(agg, degt, w, b, apply_ns, rows_out, rblk):
    return pl.pallas_call(
        functools.partial(_layer_body, apply_ns),
        grid=(rows_out // rblk,),
        in_specs=[
            pl.BlockSpec((2, rblk, F), lambda i: (0, i, 0)),
            pl.BlockSpec((rblk, 2), lambda i: (i, 0)),
            pl.BlockSpec((F, F), lambda i: (0, 0)),
            pl.BlockSpec((1, F), lambda i: (0, 0)),
        ],
        out_specs=pl.BlockSpec((rblk, F), lambda i: (i, 0)),
        out_shape=jax.ShapeDtypeStruct((rows_out, F), f32),
    )(agg, degt, w, b)


def kernel(h, edge_index, W1, b1, W2, b2):
    n_extra = NCH * CH - E
    padidx = (N + jnp.arange(n_extra, dtype=i32) % (NP - N))
    epad = jnp.concatenate(
        [edge_index, jnp.broadcast_to(padidx, (2, n_extra))], axis=1)
    ei = epad.reshape(2, NCH, CH)
    eq = epad.reshape(2, 4096, 80)

    deg_o, deg_i = _deg_call(ei)
    degt = jnp.stack([deg_o, deg_i], axis=1)

    hh1 = _tc_pre(h, degt)
    agg1 = _mp_call(hh1, eq)
    hh2 = _tc_layer(agg1, degt, W1.astype(f32), b1.reshape(1, F),
                    True, NP, _R)
    agg2 = _mp_call(hh2, eq)
    out = _tc_layer(agg2, degt, W2.astype(f32), b2.reshape(1, F),
                    False, N, 2000)
    return out


# speedup vs baseline: 1.0047x; 1.0047x over previous
import functools

import jax
import jax.numpy as jnp
from jax import lax
from jax.experimental import pallas as pl
from jax.experimental.pallas import tpu as pltpu
from jax.experimental.pallas import tpu_sc as plsc

N = 10000
NP = 10240
E = 320000
F = 128
CH = 128
NCH = 2560

f32 = jnp.float32
i32 = jnp.int32

_MESH = plsc.VectorSubcoreMesh(core_axis_name="c", subcore_axis_name="s")


def _deg_body(ei_hbm, dego_hbm, degi_hbm, slab, ones_v, zbuf, acc, sem):
    c = lax.axis_index("c")
    s = lax.axis_index("s")

    for u in range(8):
        ones_v[pl.ds(u * 16, 16)] = jnp.full((16,), 1.0, f32)
    for u in range(40):
        zbuf[pl.ds(u * 16, 16)] = jnp.zeros((16,), f32)

    pltpu.sync_copy(ei_hbm.at[c, pl.ds(s * 160, 160)], slab)
    pltpu.sync_copy(zbuf, acc.at[pl.ds(s * 640, 640)])

    plsc.subcore_barrier()

    def group(g, carry):
        for u in range(8):
            pltpu.async_copy(ones_v, acc.at[slab.at[g * 8 + u]], sem,
                             add=True)

        @pl.when(g > 0)
        def _():
            for _u in range(8):
                pltpu.make_async_copy(ones_v, acc.at[slab.at[0]],
                                      sem).wait()
        return carry

    lax.fori_loop(0, 20, group, 0)
    for _u in range(8):
        pltpu.make_async_copy(ones_v, acc.at[slab.at[0]], sem).wait()

    plsc.subcore_barrier()

    @pl.when(s < 15)
    def _():
        pltpu.sync_copy(acc.at[pl.ds(s * 640, 640)], zbuf)

        @pl.when(c == 0)
        def _():
            pltpu.sync_copy(zbuf, dego_hbm.at[pl.ds(s * 640, 640)])

        @pl.when(c == 1)
        def _():
            pltpu.sync_copy(zbuf, degi_hbm.at[pl.ds(s * 640, 640)])

    @pl.when(s == 15)
    def _():
        pltpu.sync_copy(acc.at[pl.ds(9600, 400)], zbuf.at[pl.ds(0, 400)])

        @pl.when(c == 0)
        def _():
            pltpu.sync_copy(zbuf.at[pl.ds(0, 400)],
                            dego_hbm.at[pl.ds(9600, 400)])

        @pl.when(c == 1)
        def _():
            pltpu.sync_copy(zbuf.at[pl.ds(0, 400)],
                            degi_hbm.at[pl.ds(9600, 400)])


_deg_call = functools.partial(
    pl.kernel,
    out_type=(jax.ShapeDtypeStruct((N,), f32),
              jax.ShapeDtypeStruct((N,), f32)),
    mesh=_MESH,
    scratch_types=[
        pltpu.VMEM((160, CH), i32),
        pltpu.VMEM((CH,), f32),
        pltpu.VMEM((640,), f32),
        pltpu.VMEM_SHARED((NP,), f32),
        pltpu.SemaphoreType.DMA,
    ],
)(_deg_body)


def _mp_body(hh_hbm, eq_hbm, out_hbm,
             sidx, didx, rows0, rows1, rows2, rows3,
             acc, g0, g1, g2, g3, s0, s1, s2, s3, lsem):
    c = lax.axis_index("c")
    s = lax.axis_index("s")
    q0 = c * 2048 + s * 128

    rows = (rows0, rows1, rows2, rows3)
    gsem = (g0, g1, g2, g3)
    ssem = (s0, s1, s2, s3)

    def zrow(i, carry):
        for u in range(8):
            rows0[i, pl.ds(u * 16, 16)] = jnp.zeros((16,), f32)
        return carry

    lax.fori_loop(0, 80, zrow, 0)
    for k in range(8):
        pltpu.async_copy(rows0, acc.at[pl.ds(s * 640 + k * 80, 80), :],
                         gsem[k % 4])
    for k in range(8):
        pltpu.make_async_copy(rows0, acc.at[pl.ds(0, 80), :],
                              gsem[k % 4]).wait()

    pltpu.sync_copy(eq_hbm.at[0, pl.ds(q0, 16)], sidx.at[0])
    pltpu.sync_copy(eq_hbm.at[1, pl.ds(q0, 16)], didx.at[0])
    plsc.subcore_barrier()

    def gather(slot, pb, r):
        pltpu.async_copy(hh_hbm.at[sidx.at[pb, r]], rows[slot],
                         gsem[slot])

    def scatter_chunk(u2, tm2):
        pb2 = (tm2 // 16) % 2
        r2 = tm2 % 16
        pltpu.make_async_copy(hh_hbm.at[sidx.at[0, 0]], rows[u2],
                              gsem[u2]).wait()
        pltpu.async_copy(rows[u2], acc.at[didx.at[pb2, r2]],
                         ssem[u2], add=True)

    def quad(p):
        def body(qq, carry):
            for u in range(4):
                q = qq * 4 + u
                t = p * 16 + q

                @pl.when(t >= 4)
                def _():
                    pltpu.make_async_copy(rows[u], acc.at[didx.at[0, 0]],
                                          ssem[u]).wait()

                gather(u, p % 2, q)

                @pl.when(t >= 2)
                def _():
                    scatter_chunk((u + 2) % 4, t - 2)
            return carry
        return body

    for p in range(8):
        if p > 0:
            pltpu.make_async_copy(eq_hbm.at[0, pl.ds(q0, 16)],
                                  sidx.at[p % 2], lsem).wait()
            pltpu.make_async_copy(eq_hbm.at[1, pl.ds(q0, 16)],
                                  didx.at[p % 2], lsem).wait()
        lax.fori_loop(0, 2, quad(p), 0)
        if p < 7:
            nb = (p + 1) % 2
            pltpu.async_copy(eq_hbm.at[0, pl.ds(q0 + (p + 1) * 16, 16)],
                             sidx.at[nb], lsem)
            pltpu.async_copy(eq_hbm.at[1, pl.ds(q0 + (p + 1) * 16, 16)],
                             didx.at[nb], lsem)
        lax.fori_loop(2, 4, quad(p), 0)

    for t in (128, 129):
        scatter_chunk((t - 2) % 4, t - 2)
    for u in range(4):
        pltpu.make_async_copy(rows[u], acc.at[didx.at[0, 0]],
                              ssem[u]).wait()

    plsc.subcore_barrier()
    for k in range(8):
        buf = rows[k % 4]
        if k >= 4:
            pltpu.make_async_copy(
                buf, out_hbm.at[c, pl.ds(0, 80), :], gsem[k % 4]).wait()
        pltpu.sync_copy(acc.at[pl.ds(s * 640 + k * 80, 80), :], buf)
        pltpu.async_copy(buf, out_hbm.at[c, pl.ds(s * 640 + k * 80, 80), :],
                         gsem[k % 4])
    for k in range(4):
        pltpu.make_async_copy(rows[k], out_hbm.at[c, pl.ds(0, 80), :],
                              gsem[k]).wait()


_mp_call = functools.partial(
    pl.kernel,
    out_type=jax.ShapeDtypeStruct((2, NP, F), f32),
    mesh=_MESH,
    scratch_types=[
        pltpu.VMEM((2, 16, 80), i32),
        pltpu.VMEM((2, 16, 80), i32),
        pltpu.VMEM((80, F), f32),
        pltpu.VMEM((80, F), f32),
        pltpu.VMEM((80, F), f32),
        pltpu.VMEM((80, F), f32),
        pltpu.VMEM_SHARED((NP, F), f32),
        pltpu.SemaphoreType.DMA,
        pltpu.SemaphoreType.DMA,
        pltpu.SemaphoreType.DMA,
        pltpu.SemaphoreType.DMA,
        pltpu.SemaphoreType.DMA,
        pltpu.SemaphoreType.DMA,
        pltpu.SemaphoreType.DMA,
        pltpu.SemaphoreType.DMA,
        pltpu.SemaphoreType.DMA,
    ],
)(_mp_body)


_R = 2048


def _pre_body(h_ref, degt_ref, out_ref):
    ns = lax.rsqrt(jnp.clip(degt_ref[:, 0:1], 1.0, None))
    out_ref[...] = h_ref[...] * ns


def _tc_pre(h, degt):
    return pl.pallas_call(
        _pre_body,
        grid=(NP // _R,),
        in_specs=[
            pl.BlockSpec((_R, F), lambda i: (i, 0)),
            pl.BlockSpec((_R, 2), lambda i: (i, 0)),
        ],
        out_specs=pl.BlockSpec((_R, F), lambda i: (i, 0)),
        out_shape=jax.ShapeDtypeStruct((NP, F), f32),
    )(h, degt)


def _layer_body(apply_ns, agg_ref, degt_ref, w_ref, b_ref, out_ref):
    a = agg_ref[0] + agg_ref[1]
    d = degt_ref[...]
    nd = lax.rsqrt(jnp.clip(d[:, 1:2], 1.0, None))
    x = jnp.dot(a * nd, w_ref[...], preferred_element_type=f32)
    x = jnp.maximum(x + b_ref[...], 0.0)
    if apply_ns:
        x = x * lax.rsqrt(jnp.clip(d[:, 0:1], 1.0, None))
    out_ref[...] = x


def _tc_layer(agg, degt, w, b, apply_ns, rows_out, rblk):
    return pl.pallas_call(
        functools.partial(_layer_body, apply_ns),
        grid=(rows_out // rblk,),
        in_specs=[
            pl.BlockSpec((2, rblk, F), lambda i: (0, i, 0)),
            pl.BlockSpec((rblk, 2), lambda i: (i, 0)),
            pl.BlockSpec((F, F), lambda i: (0, 0)),
            pl.BlockSpec((1, F), lambda i: (0, 0)),
        ],
        out_specs=pl.BlockSpec((rblk, F), lambda i: (i, 0)),
        out_shape=jax.ShapeDtypeStruct((rows_out, F), f32),
    )(agg, degt, w, b)


def kernel(h, edge_index, W1, b1, W2, b2):
    n_extra = NCH * CH - E
    padidx = (N + jnp.arange(n_extra, dtype=i32) % (NP - N))
    epad = jnp.concatenate(
        [edge_index, jnp.broadcast_to(padidx, (2, n_extra))], axis=1)
    ei = epad.reshape(2, NCH, CH)
    eq = epad.reshape(2, 4096, 80)

    deg_o, deg_i = _deg_call(ei)
    degt = jnp.stack([deg_o, deg_i], axis=1)

    hh1 = _tc_pre(h, degt)
    agg1 = _mp_call(hh1, eq)
    hh2 = _tc_layer(agg1, degt, W1.astype(f32), b1.reshape(1, F),
                    True, NP, _R)
    agg2 = _mp_call(hh2, eq)
    out = _tc_layer(agg2, degt, W2.astype(f32), b2.reshape(1, F),
                    False, N, 2000)
    return out


# speedup vs baseline: 1.0174x; 1.0126x over previous
import functools

import jax
import jax.numpy as jnp
from jax import lax
from jax.experimental import pallas as pl
from jax.experimental.pallas import tpu as pltpu
from jax.experimental.pallas import tpu_sc as plsc

N = 10000
NP = 10240
E = 320000
F = 128
CH = 128
NCH = 2560

f32 = jnp.float32
i32 = jnp.int32

_MESH = plsc.VectorSubcoreMesh(core_axis_name="c", subcore_axis_name="s")


def _deg_body(ei_hbm, dego_hbm, degi_hbm, slab, ones_v, zbuf, acc, sem):
    c = lax.axis_index("c")
    s = lax.axis_index("s")

    for u in range(8):
        ones_v[pl.ds(u * 16, 16)] = jnp.full((16,), 1.0, f32)
    for u in range(40):
        zbuf[pl.ds(u * 16, 16)] = jnp.zeros((16,), f32)

    @pl.when(s < 15)
    def _():
        pltpu.sync_copy(ei_hbm.at[c, pl.ds(s * 160, 160)], slab)

    @pl.when(s == 15)
    def _():
        pltpu.sync_copy(ei_hbm.at[c, pl.ds(2400, 100)],
                        slab.at[pl.ds(0, 100)])

    pltpu.sync_copy(zbuf, acc.at[pl.ds(s * 640, 640)])

    plsc.subcore_barrier()

    ng = jnp.where(s < 15, 40, 25)

    def group(g, carry):
        for u in range(4):
            pltpu.async_copy(ones_v, acc.at[slab.at[g * 4 + u]], sem,
                             add=True)

        @pl.when(g > 0)
        def _():
            for _u in range(4):
                pltpu.make_async_copy(ones_v, acc.at[slab.at[0]],
                                      sem).wait()
        return carry

    lax.fori_loop(0, ng, group, 0)
    for _u in range(4):
        pltpu.make_async_copy(ones_v, acc.at[slab.at[0]], sem).wait()

    plsc.subcore_barrier()

    @pl.when(s < 15)
    def _():
        pltpu.sync_copy(acc.at[pl.ds(s * 640, 640)], zbuf)

        @pl.when(c == 0)
        def _():
            pltpu.sync_copy(zbuf, dego_hbm.at[pl.ds(s * 640, 640)])

        @pl.when(c == 1)
        def _():
            pltpu.sync_copy(zbuf, degi_hbm.at[pl.ds(s * 640, 640)])

    @pl.when(s == 15)
    def _():
        pltpu.sync_copy(acc.at[pl.ds(9600, 400)], zbuf.at[pl.ds(0, 400)])

        @pl.when(c == 0)
        def _():
            pltpu.sync_copy(zbuf.at[pl.ds(0, 400)],
                            dego_hbm.at[pl.ds(9600, 400)])

        @pl.when(c == 1)
        def _():
            pltpu.sync_copy(zbuf.at[pl.ds(0, 400)],
                            degi_hbm.at[pl.ds(9600, 400)])


_deg_call = functools.partial(
    pl.kernel,
    out_type=(jax.ShapeDtypeStruct((N,), f32),
              jax.ShapeDtypeStruct((N,), f32)),
    mesh=_MESH,
    scratch_types=[
        pltpu.VMEM((160, CH), i32),
        pltpu.VMEM((CH,), f32),
        pltpu.VMEM((640,), f32),
        pltpu.VMEM_SHARED((NP,), f32),
        pltpu.SemaphoreType.DMA,
    ],
)(_deg_body)


def _mp_body(hh_hbm, eq_hbm, out_hbm,
             sidx, didx, rows0, rows1, rows2, rows3,
             acc, g0, g1, g2, g3, s0, s1, s2, s3, lsem):
    c = lax.axis_index("c")
    s = lax.axis_index("s")
    q0 = c * 2048 + s * 128

    rows = (rows0, rows1, rows2, rows3)
    gsem = (g0, g1, g2, g3)
    ssem = (s0, s1, s2, s3)

    def zrow(i, carry):
        for u in range(8):
            rows0[i, pl.ds(u * 16, 16)] = jnp.zeros((16,), f32)
        return carry

    lax.fori_loop(0, 80, zrow, 0)
    for k in range(8):
        pltpu.async_copy(rows0, acc.at[pl.ds(s * 640 + k * 80, 80), :],
                         gsem[k % 4])
    for k in range(8):
        pltpu.make_async_copy(rows0, acc.at[pl.ds(0, 80), :],
                              gsem[k % 4]).wait()

    pltpu.sync_copy(eq_hbm.at[0, pl.ds(q0, 16)], sidx.at[0])
    pltpu.sync_copy(eq_hbm.at[1, pl.ds(q0, 16)], didx.at[0])
    plsc.subcore_barrier()

    def gather(slot, pb, r):
        pltpu.async_copy(hh_hbm.at[sidx.at[pb, r]], rows[slot],
                         gsem[slot])

    def scatter_chunk(u2, tm2):
        pb2 = (tm2 // 16) % 2
        r2 = tm2 % 16
        pltpu.make_async_copy(hh_hbm.at[sidx.at[0, 0]], rows[u2],
                              gsem[u2]).wait()
        pltpu.async_copy(rows[u2], acc.at[didx.at[pb2, r2]],
                         ssem[u2], add=True)

    def quad(p):
        def body(qq, carry):
            for u in range(4):
                q = qq * 4 + u
                t = p * 16 + q

                @pl.when(t >= 4)
                def _():
                    pltpu.make_async_copy(rows[u], acc.at[didx.at[0, 0]],
                                          ssem[u]).wait()

                gather(u, p % 2, q)

                @pl.when(t >= 2)
                def _():
                    scatter_chunk((u + 2) % 4, t - 2)
            return carry
        return body

    for p in range(8):
        if p > 0:
            pltpu.make_async_copy(eq_hbm.at[0, pl.ds(q0, 16)],
                                  sidx.at[p % 2], lsem).wait()
            pltpu.make_async_copy(eq_hbm.at[1, pl.ds(q0, 16)],
                                  didx.at[p % 2], lsem).wait()
        lax.fori_loop(0, 2, quad(p), 0)
        if p < 7:
            nb = (p + 1) % 2
            pltpu.async_copy(eq_hbm.at[0, pl.ds(q0 + (p + 1) * 16, 16)],
                             sidx.at[nb], lsem)
            pltpu.async_copy(eq_hbm.at[1, pl.ds(q0 + (p + 1) * 16, 16)],
                             didx.at[nb], lsem)
        lax.fori_loop(2, 4, quad(p), 0)

    for t in (128, 129):
        scatter_chunk((t - 2) % 4, t - 2)
    for u in range(4):
        pltpu.make_async_copy(rows[u], acc.at[didx.at[0, 0]],
                              ssem[u]).wait()

    plsc.subcore_barrier()
    for k in range(8):
        buf = rows[k % 4]
        if k >= 4:
            pltpu.make_async_copy(
                buf, out_hbm.at[c, pl.ds(0, 80), :], gsem[k % 4]).wait()
        pltpu.sync_copy(acc.at[pl.ds(s * 640 + k * 80, 80), :], buf)
        pltpu.async_copy(buf, out_hbm.at[c, pl.ds(s * 640 + k * 80, 80), :],
                         gsem[k % 4])
    for k in range(4):
        pltpu.make_async_copy(rows[k], out_hbm.at[c, pl.ds(0, 80), :],
                              gsem[k]).wait()


_mp_call = functools.partial(
    pl.kernel,
    out_type=jax.ShapeDtypeStruct((2, NP, F), f32),
    mesh=_MESH,
    scratch_types=[
        pltpu.VMEM((2, 16, 80), i32),
        pltpu.VMEM((2, 16, 80), i32),
        pltpu.VMEM((80, F), f32),
        pltpu.VMEM((80, F), f32),
        pltpu.VMEM((80, F), f32),
        pltpu.VMEM((80, F), f32),
        pltpu.VMEM_SHARED((NP, F), f32),
        pltpu.SemaphoreType.DMA,
        pltpu.SemaphoreType.DMA,
        pltpu.SemaphoreType.DMA,
        pltpu.SemaphoreType.DMA,
        pltpu.SemaphoreType.DMA,
        pltpu.SemaphoreType.DMA,
        pltpu.SemaphoreType.DMA,
        pltpu.SemaphoreType.DMA,
        pltpu.SemaphoreType.DMA,
    ],
)(_mp_body)


_R = 2048


def _pre_body(h_ref, degt_ref, out_ref):
    ns = lax.rsqrt(jnp.clip(degt_ref[:, 0:1], 1.0, None))
    out_ref[...] = h_ref[...] * ns


def _tc_pre(h, degt):
    return pl.pallas_call(
        _pre_body,
        grid=(NP // _R,),
        in_specs=[
            pl.BlockSpec((_R, F), lambda i: (i, 0)),
            pl.BlockSpec((_R, 2), lambda i: (i, 0)),
        ],
        out_specs=pl.BlockSpec((_R, F), lambda i: (i, 0)),
        out_shape=jax.ShapeDtypeStruct((NP, F), f32),
    )(h, degt)


def _layer_body(apply_ns, agg_ref, degt_ref, w_ref, b_ref, out_ref):
    a = agg_ref[0] + agg_ref[1]
    d = degt_ref[...]
    nd = lax.rsqrt(jnp.clip(d[:, 1:2], 1.0, None))
    x = jnp.dot(a * nd, w_ref[...], preferred_element_type=f32)
    x = jnp.maximum(x + b_ref[...], 0.0)
    if apply_ns:
        x = x * lax.rsqrt(jnp.clip(d[:, 0:1], 1.0, None))
    out_ref[...] = x


def _tc_layer(agg, degt, w, b, apply_ns, rows_out, rblk):
    return pl.pallas_call(
        functools.partial(_layer_body, apply_ns),
        grid=(rows_out // rblk,),
        in_specs=[
            pl.BlockSpec((2, rblk, F), lambda i: (0, i, 0)),
            pl.BlockSpec((rblk, 2), lambda i: (i, 0)),
            pl.BlockSpec((F, F), lambda i: (0, 0)),
            pl.BlockSpec((1, F), lambda i: (0, 0)),
        ],
        out_specs=pl.BlockSpec((rblk, F), lambda i: (i, 0)),
        out_shape=jax.ShapeDtypeStruct((rows_out, F), f32),
    )(agg, degt, w, b)


def kernel(h, edge_index, W1, b1, W2, b2):
    n_extra = NCH * CH - E
    padidx = (N + jnp.arange(n_extra, dtype=i32) % (NP - N))
    epad = jnp.concatenate(
        [edge_index, jnp.broadcast_to(padidx, (2, n_extra))], axis=1)
    ei = edge_index.reshape(2, 2500, CH)
    eq = epad.reshape(2, 4096, 80)

    deg_o, deg_i = _deg_call(ei)
    degt = jnp.stack([deg_o, deg_i], axis=1)

    hh1 = _tc_pre(h, degt)
    agg1 = _mp_call(hh1, eq)
    hh2 = _tc_layer(agg1, degt, W1.astype(f32), b1.reshape(1, F),
                    True, NP, _R)
    agg2 = _mp_call(hh2, eq)
    out = _tc_layer(agg2, degt, W2.astype(f32), b2.reshape(1, F),
                    False, N, 2000)
    return out


# speedup vs baseline: 1.0290x; 1.0114x over previous
import functools

import jax
import jax.numpy as jnp
from jax import lax
from jax.experimental import pallas as pl
from jax.experimental.pallas import tpu as pltpu
from jax.experimental.pallas import tpu_sc as plsc

N = 10000
NP = 10240
E = 320000
F = 128
CH = 128
NCH = 2560

f32 = jnp.float32
i32 = jnp.int32

_MESH = plsc.VectorSubcoreMesh(core_axis_name="c", subcore_axis_name="s")


def _deg_body(ei_hbm, dego_hbm, degi_hbm, slab, ones_v, zbuf, acc, sem):
    c = lax.axis_index("c")
    s = lax.axis_index("s")

    for u in range(8):
        ones_v[pl.ds(u * 16, 16)] = jnp.full((16,), 1.0, f32)
    for u in range(40):
        zbuf[pl.ds(u * 16, 16)] = jnp.zeros((16,), f32)

    @pl.when(s < 15)
    def _():
        pltpu.sync_copy(ei_hbm.at[c, pl.ds(s * 160, 160)], slab)

    @pl.when(s == 15)
    def _():
        pltpu.sync_copy(ei_hbm.at[c, pl.ds(2400, 100)],
                        slab.at[pl.ds(0, 100)])

    pltpu.sync_copy(zbuf, acc.at[pl.ds(s * 640, 640)])

    plsc.subcore_barrier()

    ng = jnp.where(s < 15, 40, 25)

    def group(g, carry):
        for u in range(4):
            pltpu.async_copy(ones_v, acc.at[slab.at[g * 4 + u]], sem,
                             add=True)

        @pl.when(g > 0)
        def _():
            for _u in range(4):
                pltpu.make_async_copy(ones_v, acc.at[slab.at[0]],
                                      sem).wait()
        return carry

    lax.fori_loop(0, ng, group, 0)
    for _u in range(4):
        pltpu.make_async_copy(ones_v, acc.at[slab.at[0]], sem).wait()

    plsc.subcore_barrier()

    @pl.when(s < 15)
    def _():
        pltpu.sync_copy(acc.at[pl.ds(s * 640, 640)], zbuf)

        @pl.when(c == 0)
        def _():
            pltpu.sync_copy(zbuf, dego_hbm.at[pl.ds(s * 640, 640)])

        @pl.when(c == 1)
        def _():
            pltpu.sync_copy(zbuf, degi_hbm.at[pl.ds(s * 640, 640)])

    @pl.when(s == 15)
    def _():
        pltpu.sync_copy(acc.at[pl.ds(9600, 400)], zbuf.at[pl.ds(0, 400)])

        @pl.when(c == 0)
        def _():
            pltpu.sync_copy(zbuf.at[pl.ds(0, 400)],
                            dego_hbm.at[pl.ds(9600, 400)])

        @pl.when(c == 1)
        def _():
            pltpu.sync_copy(zbuf.at[pl.ds(0, 400)],
                            degi_hbm.at[pl.ds(9600, 400)])


_deg_call = functools.partial(
    pl.kernel,
    out_type=(jax.ShapeDtypeStruct((N,), f32),
              jax.ShapeDtypeStruct((N,), f32)),
    mesh=_MESH,
    scratch_types=[
        pltpu.VMEM((160, CH), i32),
        pltpu.VMEM((CH,), f32),
        pltpu.VMEM((640,), f32),
        pltpu.VMEM_SHARED((NP,), f32),
        pltpu.SemaphoreType.DMA,
    ],
)(_deg_body)


def _mp_body(hh_hbm, eq_hbm, out_hbm,
             sidx, didx, rows0, rows1, rows2, rows3,
             acc, g0, g1, g2, g3, s0, s1, s2, s3, lsem):
    c = lax.axis_index("c")
    s = lax.axis_index("s")
    q0 = c * 2048 + s * 128

    rows = (rows0, rows1, rows2, rows3)
    gsem = (g0, g1, g2, g3)
    ssem = (s0, s1, s2, s3)

    def zrow(i, carry):
        for u in range(8):
            rows0[i, pl.ds(u * 16, 16)] = jnp.zeros((16,), f32)
        return carry

    pltpu.async_copy(eq_hbm.at[0, pl.ds(q0, 16)], sidx.at[0], lsem)
    pltpu.async_copy(eq_hbm.at[1, pl.ds(q0, 16)], didx.at[0], lsem)

    lax.fori_loop(0, 80, zrow, 0)
    for k in range(8):
        pltpu.async_copy(rows0, acc.at[pl.ds(s * 640 + k * 80, 80), :],
                         gsem[k % 4])
    for k in range(8):
        pltpu.make_async_copy(rows0, acc.at[pl.ds(0, 80), :],
                              gsem[k % 4]).wait()
    pltpu.make_async_copy(eq_hbm.at[0, pl.ds(q0, 16)], sidx.at[0],
                          lsem).wait()
    pltpu.make_async_copy(eq_hbm.at[1, pl.ds(q0, 16)], didx.at[0],
                          lsem).wait()
    plsc.subcore_barrier()

    def gather(slot, pb, r):
        pltpu.async_copy(hh_hbm.at[sidx.at[pb, r]], rows[slot],
                         gsem[slot])

    def scatter_chunk(u2, tm2):
        pb2 = (tm2 // 16) % 2
        r2 = tm2 % 16
        pltpu.make_async_copy(hh_hbm.at[sidx.at[0, 0]], rows[u2],
                              gsem[u2]).wait()
        pltpu.async_copy(rows[u2], acc.at[didx.at[pb2, r2]],
                         ssem[u2], add=True)

    def quad(p):
        def body(qq, carry):
            for u in range(4):
                q = qq * 4 + u
                t = p * 16 + q

                @pl.when(t >= 4)
                def _():
                    pltpu.make_async_copy(rows[u], acc.at[didx.at[0, 0]],
                                          ssem[u]).wait()

                gather(u, p % 2, q)

                @pl.when(t >= 2)
                def _():
                    scatter_chunk((u + 2) % 4, t - 2)
            return carry
        return body

    for p in range(8):
        if p > 0:
            pltpu.make_async_copy(eq_hbm.at[0, pl.ds(q0, 16)],
                                  sidx.at[p % 2], lsem).wait()
            pltpu.make_async_copy(eq_hbm.at[1, pl.ds(q0, 16)],
                                  didx.at[p % 2], lsem).wait()
        lax.fori_loop(0, 2, quad(p), 0)
        if p < 7:
            nb = (p + 1) % 2
            pltpu.async_copy(eq_hbm.at[0, pl.ds(q0 + (p + 1) * 16, 16)],
                             sidx.at[nb], lsem)
            pltpu.async_copy(eq_hbm.at[1, pl.ds(q0 + (p + 1) * 16, 16)],
                             didx.at[nb], lsem)
        lax.fori_loop(2, 4, quad(p), 0)

    for t in (128, 129):
        scatter_chunk((t - 2) % 4, t - 2)
    for u in range(4):
        pltpu.make_async_copy(rows[u], acc.at[didx.at[0, 0]],
                              ssem[u]).wait()

    plsc.subcore_barrier()
    for k in range(8):
        buf = rows[k % 4]
        if k >= 4:
            pltpu.make_async_copy(
                buf, out_hbm.at[c, pl.ds(0, 80), :], gsem[k % 4]).wait()
        pltpu.sync_copy(acc.at[pl.ds(s * 640 + k * 80, 80), :], buf)
        pltpu.async_copy(buf, out_hbm.at[c, pl.ds(s * 640 + k * 80, 80), :],
                         gsem[k % 4])
    for k in range(4):
        pltpu.make_async_copy(rows[k], out_hbm.at[c, pl.ds(0, 80), :],
                              gsem[k]).wait()


_mp_call = functools.partial(
    pl.kernel,
    out_type=jax.ShapeDtypeStruct((2, NP, F), f32),
    mesh=_MESH,
    scratch_types=[
        pltpu.VMEM((2, 16, 80), i32),
        pltpu.VMEM((2, 16, 80), i32),
        pltpu.VMEM((80, F), f32),
        pltpu.VMEM((80, F), f32),
        pltpu.VMEM((80, F), f32),
        pltpu.VMEM((80, F), f32),
        pltpu.VMEM_SHARED((NP, F), f32),
        pltpu.SemaphoreType.DMA,
        pltpu.SemaphoreType.DMA,
        pltpu.SemaphoreType.DMA,
        pltpu.SemaphoreType.DMA,
        pltpu.SemaphoreType.DMA,
        pltpu.SemaphoreType.DMA,
        pltpu.SemaphoreType.DMA,
        pltpu.SemaphoreType.DMA,
        pltpu.SemaphoreType.DMA,
    ],
)(_mp_body)


_R = 2048


def _pre_body(h_ref, degt_ref, out_ref):
    ns = lax.rsqrt(jnp.clip(degt_ref[:, 0:1], 1.0, None))
    out_ref[...] = h_ref[...] * ns


def _tc_pre(h, degt):
    return pl.pallas_call(
        _pre_body,
        grid=(NP // _R,),
        in_specs=[
            pl.BlockSpec((_R, F), lambda i: (i, 0)),
            pl.BlockSpec((_R, 2), lambda i: (i, 0)),
        ],
        out_specs=pl.BlockSpec((_R, F), lambda i: (i, 0)),
        out_shape=jax.ShapeDtypeStruct((NP, F), f32),
    )(h, degt)


def _layer_body(apply_ns, agg_ref, degt_ref, w_ref, b_ref, out_ref):
    a = agg_ref[0] + agg_ref[1]
    d = degt_ref[...]
    nd = lax.rsqrt(jnp.clip(d[:, 1:2], 1.0, None))
    x = jnp.dot(a * nd, w_ref[...], preferred_element_type=f32)
    x = jnp.maximum(x + b_ref[...], 0.0)
    if apply_ns:
        x = x * lax.rsqrt(jnp.clip(d[:, 0:1], 1.0, None))
    out_ref[...] = x


def _tc_layer(agg, degt, w, b, apply_ns, rows_out, rblk):
    return pl.pallas_call(
        functools.partial(_layer_body, apply_ns),
        grid=(rows_out // rblk,),
        in_specs=[
            pl.BlockSpec((2, rblk, F), lambda i: (0, i, 0)),
            pl.BlockSpec((rblk, 2), lambda i: (i, 0)),
            pl.BlockSpec((F, F), lambda i: (0, 0)),
            pl.BlockSpec((1, F), lambda i: (0, 0)),
        ],
        out_specs=pl.BlockSpec((rblk, F), lambda i: (i, 0)),
        out_shape=jax.ShapeDtypeStruct((rows_out, F), f32),
    )(agg, degt, w, b)


def kernel(h, edge_index, W1, b1, W2, b2):
    n_extra = NCH * CH - E
    padidx = (N + jnp.arange(n_extra, dtype=i32) % (NP - N))
    epad = jnp.concatenate(
        [edge_index, jnp.broadcast_to(padidx, (2, n_extra))], axis=1)
    ei = edge_index.reshape(2, 2500, CH)
    eq = epad.reshape(2, 4096, 80)

    deg_o, deg_i = _deg_call(ei)
    degt = jnp.stack([deg_o, deg_i], axis=1)

    hh1 = _tc_pre(h, degt)
    agg1 = _mp_call(hh1, eq)
    hh2 = _tc_layer(agg1, degt, W1.astype(f32), b1.reshape(1, F),
                    True, NP, _R)
    agg2 = _mp_call(hh2, eq)
    out = _tc_layer(agg2, degt, W2.astype(f32), b2.reshape(1, F),
                    False, N, 2000)
    return out
